# Initial kernel scaffold; baseline (speedup 1.0000x reference)
import jax
import jax.numpy as jnp
from jax.experimental import pallas as pl


def kernel(x, edge_index, edge_features, Wq, bq, Wk, bk, Wv, bv, We, be, Wox, box, Woe, boe, ln_x_g, ln_x_b, ln_e_g, ln_e_b, ffx_ln_g, ffx_ln_b, ffx_W1, ffx_b1, ffx_W2, ffx_b2, ffe_ln_g, ffe_ln_b, ffe_W1, ffe_b1, ffe_W2, ffe_b2):
    raise NotImplementedError("write your pallas kernel here")


# speedup vs baseline: 19.6347x; 19.6347x over previous
import functools

import jax
import jax.numpy as jnp
from jax import lax
from jax.experimental import pallas as pl
from jax.experimental.pallas import tpu as pltpu
from jax.experimental.pallas import tpu_sc as plsc

N = 10000
E = 320000
D = 128
DE = 16
H = 8
DH = 16
DFF = 512
DEFF = 64
EPS = 1e-05

NPAD = 10240
NPAD2 = 11520
NW = 32
EPW = E // NW
CH = 80
NCH = EPW // CH
DW = 2 * D
NG = E // CH


def _ln(x, g, b):
    mu = jnp.mean(x, axis=-1, keepdims=True)
    xc = x - mu
    var = jnp.mean(xc * xc, axis=-1, keepdims=True)
    return xc * lax.rsqrt(var + EPS) * g + b


def _gelu(x):
    return x * 0.5 * (1.0 + lax.erf(x * 0.7071067811865476))


def _tc1_body(x_ref, g_ref, b_ref, wq_ref, bq_ref, wk_ref, bk_ref,
              wv_ref, bv_ref, q_ref, kv_ref):
    xn = _ln(x_ref[...], g_ref[...], b_ref[...])
    q_ref[...] = jnp.dot(xn, wq_ref[...], preferred_element_type=jnp.float32) + bq_ref[...]
    kv_ref[:, :D] = jnp.dot(xn, wk_ref[...], preferred_element_type=jnp.float32) + bk_ref[...]
    kv_ref[:, D:] = jnp.dot(xn, wv_ref[...], preferred_element_type=jnp.float32) + bv_ref[...]


def _tc1(x, ln_x_g, ln_x_b, Wq, bq, Wk, bk, Wv, bv):
    BN = 2000
    grid = N // BN
    full = lambda shape: pl.BlockSpec(shape, lambda i: (0, 0))
    return pl.pallas_call(
        _tc1_body,
        grid=(grid,),
        in_specs=[
            pl.BlockSpec((BN, D), lambda i: (i, 0)),
            full((1, D)), full((1, D)),
            full((D, D)), full((1, D)),
            full((D, D)), full((1, D)),
            full((D, D)), full((1, D)),
        ],
        out_specs=[
            pl.BlockSpec((BN, D), lambda i: (i, 0)),
            pl.BlockSpec((BN, 2 * D), lambda i: (i, 0)),
        ],
        out_shape=[
            jax.ShapeDtypeStruct((N, D), jnp.float32),
            jax.ShapeDtypeStruct((N, 2 * D), jnp.float32),
        ],
    )(x, ln_x_g, ln_x_b, Wq, bq, Wk, bk, Wv, bv)


def _sc1_body(dst_ref, src_ref, q_ref, kv_ref, qd_ref, kvs_ref,
              idxd, idxs, qbuf, kvbuf, sem1, sem2):
    wid = lax.axis_index("s") * 2 + lax.axis_index("c")
    base = pl.multiple_of(wid * EPW, 8)

    def chunk(i, carry):
        off = pl.multiple_of(base + i * CH, 8)
        pltpu.sync_copy(dst_ref.at[pl.ds(off, CH)], idxd)
        pltpu.sync_copy(src_ref.at[pl.ds(off, CH)], idxs)
        c1 = pltpu.async_copy(q_ref.at[idxd], qbuf, sem1)
        c2 = pltpu.async_copy(kv_ref.at[idxs], kvbuf, sem2)
        c1.wait()
        c2.wait()
        pltpu.sync_copy(qbuf, qd_ref.at[pl.ds(off, CH)])
        pltpu.sync_copy(kvbuf, kvs_ref.at[pl.ds(off, CH)])
        return carry

    lax.fori_loop(0, NCH, chunk, 0)


def _sc1(dst, src, q, kv):
    mesh = plsc.VectorSubcoreMesh(core_axis_name="c", subcore_axis_name="s")
    f = functools.partial(
        pl.kernel,
        mesh=mesh,
        out_type=[
            jax.ShapeDtypeStruct((E, D), jnp.float32),
            jax.ShapeDtypeStruct((E, 2 * D), jnp.float32),
        ],
        scratch_types=[
            pltpu.VMEM((CH,), jnp.int32),
            pltpu.VMEM((CH,), jnp.int32),
            pltpu.VMEM((CH, D), jnp.float32),
            pltpu.VMEM((CH, 2 * D), jnp.float32),
            pltpu.SemaphoreType.DMA,
            pltpu.SemaphoreType.DMA,
        ],
    )(_sc1_body)
    return f(dst, src, q, kv)


def _tc2_body(qd_ref, kvs_ref, ef_ref, dst_ref, we_ref, be_ref, woe_ref, boe_ref,
              lng_ref, lnb_ref, sel_ref, selt_ref,
              fg_ref, fb_ref, w1_ref, b1_ref, w2_ref, b2_ref,
              eout_ref, wva_ref, dst2_ref):
    ks = kvs_ref[:, :D]
    vs = kvs_ref[:, D:]
    ef = ef_ref[...]
    en = _ln(ef, lng_ref[...], lnb_ref[...])
    ep = jnp.dot(en, we_ref[...], preferred_element_type=jnp.float32) + be_ref[...]
    score = qd_ref[...] * ks * ep * 0.25
    e_att = jnp.dot(score, woe_ref[...], preferred_element_type=jnp.float32) + boe_ref[...]
    logits = jnp.dot(score, sel_ref[...], preferred_element_type=jnp.float32)
    a = jnp.exp(logits)
    ab = jnp.dot(a, selt_ref[...], preferred_element_type=jnp.float32)
    wva = jnp.concatenate([ab * vs, ab], axis=1)

    ng = wva.shape[0] // CH
    d3 = dst_ref[...].reshape(-1, CH)
    eq = (d3[:, :, None] == d3[:, None, :])
    eqf = eq.astype(jnp.float32)
    ii = lax.broadcasted_iota(jnp.int32, (CH, CH), 0)
    jj = lax.broadcasted_iota(jnp.int32, (CH, CH), 1)
    tm = (jj < ii).astype(jnp.float32)
    before = jnp.sum(eqf * tm[None], axis=2)
    first = before == 0.0
    rows3 = wva.reshape(ng, CH, DW)
    comb = lax.dot_general(eqf, rows3, (((2,), (1,)), ((0,), (0,))),
                           preferred_element_type=jnp.float32)
    rows3 = jnp.where(first[:, :, None], comb, 0.0)
    wva_ref[...] = rows3.reshape(ng * CH, DW)
    dst2_ref[...] = jnp.where(first, d3, NPAD).reshape(ng, 1, CH)

    e1 = e_att + ef
    h = _ln(e1, fg_ref[...], fb_ref[...])
    h = _gelu(jnp.dot(h, w1_ref[...], preferred_element_type=jnp.float32) + b1_ref[...])
    eout_ref[...] = jnp.dot(h, w2_ref[...], preferred_element_type=jnp.float32) + b2_ref[...] + e1


def _tc2(qd, kvs, ef, dst3, We, be, Woe, boe, ln_e_g, ln_e_b, sel, selt,
         ffe_ln_g, ffe_ln_b, ffe_W1, ffe_b1, ffe_W2, ffe_b2):
    BE = 2000
    GB = BE // CH
    grid = E // BE
    full = lambda shape: pl.BlockSpec(shape, lambda i: (0, 0))
    return pl.pallas_call(
        _tc2_body,
        grid=(grid,),
        in_specs=[
            pl.BlockSpec((BE, D), lambda i: (i, 0)),
            pl.BlockSpec((BE, 2 * D), lambda i: (i, 0)),
            pl.BlockSpec((BE, DE), lambda i: (i, 0)),
            pl.BlockSpec((GB, 1, CH), lambda i: (i, 0, 0)),
            full((DE, D)), full((1, D)),
            full((D, DE)), full((1, DE)),
            full((1, DE)), full((1, DE)),
            full((D, H)), full((H, D)),
            full((1, DE)), full((1, DE)),
            full((DE, DEFF)), full((1, DEFF)),
            full((DEFF, DE)), full((1, DE)),
        ],
        out_specs=[
            pl.BlockSpec((BE, DE), lambda i: (i, 0)),
            pl.BlockSpec((BE, DW), lambda i: (i, 0)),
            pl.BlockSpec((GB, 1, CH), lambda i: (i, 0, 0)),
        ],
        out_shape=[
            jax.ShapeDtypeStruct((E, DE), jnp.float32),
            jax.ShapeDtypeStruct((E, DW), jnp.float32),
            jax.ShapeDtypeStruct((NG, 1, CH), jnp.int32),
        ],
    )(qd, kvs, ef, dst3, We, be, Woe, boe, ln_e_g, ln_e_b, sel, selt,
      ffe_ln_g, ffe_ln_b, ffe_W1, ffe_b1, ffe_W2, ffe_b2)


def _sc3_body(dst2_ref, wva_ref, zin_ref, oacc_ref, idxbuf, rowbuf, accbuf,
              zbuf, sem1):
    c = lax.axis_index("c")
    s = lax.axis_index("s")
    wid = s * 2 + c
    base = pl.multiple_of(wid * EPW, 8)
    gbase = wid * NCH
    pbase = wid * NPAD2

    pltpu.sync_copy(zin_ref, zbuf)

    def zrow(i, carry):
        pltpu.sync_copy(zbuf, oacc_ref.at[pl.ds(pbase + i * CH, CH)])
        return carry
    lax.fori_loop(0, NPAD2 // CH, zrow, 0)

    def chunk(i, carry):
        off = pl.multiple_of(base + i * CH, 8)
        pltpu.sync_copy(dst2_ref.at[pl.ds(off, CH)], idxbuf)
        pltpu.sync_copy(wva_ref.at[pl.ds(off, CH)], rowbuf)
        for k in range(CH // 16):
            idxbuf[pl.ds(k * 16, 16)] = idxbuf[pl.ds(k * 16, 16)] + pbase
        pltpu.async_copy(oacc_ref.at[idxbuf], accbuf, sem1).wait()

        def rmw(r, cc):
            for j in range(DW // 16):
                accbuf[r, pl.ds(j * 16, 16)] = (
                    accbuf[r, pl.ds(j * 16, 16)] + rowbuf[r, pl.ds(j * 16, 16)])
            return cc
        lax.fori_loop(0, CH, rmw, 0)

        pltpu.sync_copy(accbuf, oacc_ref.at[idxbuf])
        return carry

    lax.fori_loop(0, NCH, chunk, 0)


def _sc3(dst2, wva, zin):
    mesh = plsc.VectorSubcoreMesh(core_axis_name="c", subcore_axis_name="s")
    f = functools.partial(
        pl.kernel,
        mesh=mesh,
        out_type=jax.ShapeDtypeStruct((NW * NPAD2, DW), jnp.float32),
        scratch_types=[
            pltpu.VMEM((CH,), jnp.int32),
            pltpu.VMEM((CH, DW), jnp.float32),
            pltpu.VMEM((CH, DW), jnp.float32),
            pltpu.VMEM((CH, DW), jnp.float32),
            pltpu.SemaphoreType.DMA,
        ],
    )(_sc3_body)
    return f(dst2, wva, zin)


def _tc2b_body(p_ref, acc_ref, accsc):
    w = pl.program_id(1)

    @pl.when(w == 0)
    def _():
        accsc[...] = p_ref[...]

    @pl.when(w > 0)
    def _():
        accsc[...] = accsc[...] + p_ref[...]

    @pl.when(w == NW - 1)
    def _():
        acc_ref[...] = accsc[...]


def _tc2b(parts):
    BN = 1280
    grid_i = NPAD // BN
    stride = NPAD2 // BN
    return pl.pallas_call(
        _tc2b_body,
        grid=(grid_i, NW),
        in_specs=[pl.BlockSpec((BN, DW), lambda i, w: (w * stride + i, 0))],
        out_specs=pl.BlockSpec((BN, DW), lambda i, w: (i, 0)),
        out_shape=jax.ShapeDtypeStruct((NPAD, DW), jnp.float32),
        scratch_shapes=[pltpu.VMEM((BN, DW), jnp.float32)],
    )(parts)


def _tc3_body(x_ref, acc_ref, wox_ref, box_ref, lng_ref, lnb_ref,
              w1_ref, b1_ref, w2_ref, b2_ref, out_ref):
    agg = acc_ref[:, :D]
    den = acc_ref[:, D:]
    xa = agg / (den + 1e-9)
    x_att = jnp.dot(xa, wox_ref[...], preferred_element_type=jnp.float32) + box_ref[...]
    x1 = x_att + x_ref[...]
    h = _ln(x1, lng_ref[...], lnb_ref[...])
    h = _gelu(jnp.dot(h, w1_ref[...], preferred_element_type=jnp.float32) + b1_ref[...])
    out_ref[...] = jnp.dot(h, w2_ref[...], preferred_element_type=jnp.float32) + b2_ref[...] + x1


def _tc3(xpad, acc, Wox, box, ffx_ln_g, ffx_ln_b, ffx_W1, ffx_b1, ffx_W2, ffx_b2):
    BN = 1280
    grid = NPAD // BN
    full = lambda shape: pl.BlockSpec(shape, lambda i: (0, 0))
    return pl.pallas_call(
        _tc3_body,
        grid=(grid,),
        in_specs=[
            pl.BlockSpec((BN, D), lambda i: (i, 0)),
            pl.BlockSpec((BN, DW), lambda i: (i, 0)),
            full((D, D)), full((1, D)),
            full((1, D)), full((1, D)),
            full((D, DFF)), full((1, DFF)),
            full((DFF, D)), full((1, D)),
        ],
        out_specs=pl.BlockSpec((BN, D), lambda i: (i, 0)),
        out_shape=jax.ShapeDtypeStruct((NPAD, D), jnp.float32),
    )(xpad, acc, Wox, box, ffx_ln_g, ffx_ln_b, ffx_W1, ffx_b1, ffx_W2, ffx_b2)


def kernel(x, edge_index, edge_features, Wq, bq, Wk, bk, Wv, bv, We, be,
           Wox, box, Woe, boe, ln_x_g, ln_x_b, ln_e_g, ln_e_b,
           ffx_ln_g, ffx_ln_b, ffx_W1, ffx_b1, ffx_W2, ffx_b2,
           ffe_ln_g, ffe_ln_b, ffe_W1, ffe_b1, ffe_W2, ffe_b2):
    src = edge_index[0]
    dst = edge_index[1]
    r = lambda v: v.reshape(1, -1)

    hd = jnp.arange(D, dtype=jnp.int32) // DH
    sel = (hd[:, None] == jnp.arange(H, dtype=jnp.int32)[None, :]).astype(jnp.float32)
    selt = sel.T

    q, kv = _tc1(x, r(ln_x_g), r(ln_x_b), Wq, r(bq), Wk, r(bk), Wv, r(bv))
    qd, kvs = _sc1(dst, src, q, kv)
    e_out, wva, dst2 = _tc2(
        qd, kvs, edge_features, dst.reshape(NG, 1, CH), We, r(be), Woe, r(boe),
        r(ln_e_g), r(ln_e_b), sel, selt,
        r(ffe_ln_g), r(ffe_ln_b), ffe_W1, r(ffe_b1), ffe_W2, r(ffe_b2))
    parts = _sc3(dst2.reshape(E), wva, jnp.zeros((CH, DW), jnp.float32))
    acc = _tc2b(parts)
    xpad = jnp.zeros((NPAD, D), jnp.float32).at[:N].set(x)
    x_out = _tc3(
        xpad, acc, Wox, r(box), r(ffx_ln_g), r(ffx_ln_b),
        ffx_W1, r(ffx_b1), ffx_W2, r(ffx_b2))[:N]
    return (x_out, e_out)


# speedup vs baseline: 22.0470x; 1.1229x over previous
import functools

import jax
import jax.numpy as jnp
from jax import lax
from jax.experimental import pallas as pl
from jax.experimental.pallas import tpu as pltpu
from jax.experimental.pallas import tpu_sc as plsc

N = 10000
E = 320000
D = 128
DE = 16
H = 8
DH = 16
DFF = 512
DEFF = 64
EPS = 1e-05

NPAD = 10240
NPAD2 = 11520
NW = 32
EPW = E // NW
CH = 80
NCH = EPW // CH
DW = 2 * D
NG = E // CH


def _ln(x, g, b):
    mu = jnp.mean(x, axis=-1, keepdims=True)
    xc = x - mu
    var = jnp.mean(xc * xc, axis=-1, keepdims=True)
    return xc * lax.rsqrt(var + EPS) * g + b


def _gelu(x):
    return x * 0.5 * (1.0 + lax.erf(x * 0.7071067811865476))


def _tc1_body(x_ref, g_ref, b_ref, wq_ref, bq_ref, wk_ref, bk_ref,
              wv_ref, bv_ref, q_ref, kv_ref):
    xn = _ln(x_ref[...], g_ref[...], b_ref[...])
    q_ref[...] = jnp.dot(xn, wq_ref[...], preferred_element_type=jnp.float32) + bq_ref[...]
    kv_ref[:, :D] = jnp.dot(xn, wk_ref[...], preferred_element_type=jnp.float32) + bk_ref[...]
    kv_ref[:, D:] = jnp.dot(xn, wv_ref[...], preferred_element_type=jnp.float32) + bv_ref[...]


def _tc1(x, ln_x_g, ln_x_b, Wq, bq, Wk, bk, Wv, bv):
    BN = 2000
    grid = N // BN
    full = lambda shape: pl.BlockSpec(shape, lambda i: (0, 0))
    return pl.pallas_call(
        _tc1_body,
        grid=(grid,),
        in_specs=[
            pl.BlockSpec((BN, D), lambda i: (i, 0)),
            full((1, D)), full((1, D)),
            full((D, D)), full((1, D)),
            full((D, D)), full((1, D)),
            full((D, D)), full((1, D)),
        ],
        out_specs=[
            pl.BlockSpec((BN, D), lambda i: (i, 0)),
            pl.BlockSpec((BN, 2 * D), lambda i: (i, 0)),
        ],
        out_shape=[
            jax.ShapeDtypeStruct((N, D), jnp.float32),
            jax.ShapeDtypeStruct((N, 2 * D), jnp.float32),
        ],
    )(x, ln_x_g, ln_x_b, Wq, bq, Wk, bk, Wv, bv)


def _sc1_body(dst_ref, src_ref, q_ref, kv_ref, qd_ref, kvs_ref,
              idxd, idxs, qbuf, kvbuf, sem1, sem2):
    wid = lax.axis_index("s") * 2 + lax.axis_index("c")
    base = pl.multiple_of(wid * EPW, 8)

    pltpu.sync_copy(dst_ref.at[pl.ds(base, EPW)], idxd)
    pltpu.sync_copy(src_ref.at[pl.ds(base, EPW)], idxs)

    def chunk(i, carry):
        off = pl.multiple_of(base + i * CH, 8)
        c1 = pltpu.async_copy(q_ref.at[idxd.at[pl.ds(i * CH, CH)]], qbuf, sem1)
        c2 = pltpu.async_copy(kv_ref.at[idxs.at[pl.ds(i * CH, CH)]], kvbuf, sem2)
        c1.wait()
        c2.wait()
        pltpu.sync_copy(qbuf, qd_ref.at[pl.ds(off, CH)])
        pltpu.sync_copy(kvbuf, kvs_ref.at[pl.ds(off, CH)])
        return carry

    lax.fori_loop(0, NCH, chunk, 0)


def _sc1(dst, src, q, kv):
    mesh = plsc.VectorSubcoreMesh(core_axis_name="c", subcore_axis_name="s")
    f = functools.partial(
        pl.kernel,
        mesh=mesh,
        out_type=[
            jax.ShapeDtypeStruct((E, D), jnp.float32),
            jax.ShapeDtypeStruct((E, 2 * D), jnp.float32),
        ],
        scratch_types=[
            pltpu.VMEM((EPW,), jnp.int32),
            pltpu.VMEM((EPW,), jnp.int32),
            pltpu.VMEM((CH, D), jnp.float32),
            pltpu.VMEM((CH, 2 * D), jnp.float32),
            pltpu.SemaphoreType.DMA,
            pltpu.SemaphoreType.DMA,
        ],
    )(_sc1_body)
    return f(dst, src, q, kv)


def _tc2_body(qd_ref, kvs_ref, ef_ref, dst_ref, we_ref, be_ref, woe_ref, boe_ref,
              lng_ref, lnb_ref, sel_ref, selt_ref,
              fg_ref, fb_ref, w1_ref, b1_ref, w2_ref, b2_ref,
              eout_ref, wva_ref, dst2_ref):
    ks = kvs_ref[:, :D]
    vs = kvs_ref[:, D:]
    ef = ef_ref[...]
    en = _ln(ef, lng_ref[...], lnb_ref[...])
    ep = jnp.dot(en, we_ref[...], preferred_element_type=jnp.float32) + be_ref[...]
    score = qd_ref[...] * ks * ep * 0.25
    e_att = jnp.dot(score, woe_ref[...], preferred_element_type=jnp.float32) + boe_ref[...]
    logits = jnp.dot(score, sel_ref[...], preferred_element_type=jnp.float32)
    a = jnp.exp(logits)
    ab = jnp.dot(a, selt_ref[...], preferred_element_type=jnp.float32)
    wva = jnp.concatenate([ab * vs, ab], axis=1)

    ng = wva.shape[0] // CH
    d3 = dst_ref[...].reshape(-1, CH)
    eq = (d3[:, :, None] == d3[:, None, :])
    eqf = eq.astype(jnp.float32)
    ii = lax.broadcasted_iota(jnp.int32, (CH, CH), 0)
    jj = lax.broadcasted_iota(jnp.int32, (CH, CH), 1)
    tm = (jj < ii).astype(jnp.float32)
    before = jnp.sum(eqf * tm[None], axis=2)
    first = before == 0.0
    rows3 = wva.reshape(ng, CH, DW)
    comb = lax.dot_general(eqf, rows3, (((2,), (1,)), ((0,), (0,))),
                           preferred_element_type=jnp.float32)
    rows3 = jnp.where(first[:, :, None], comb, 0.0)
    wva_ref[...] = rows3.reshape(ng * CH, DW)
    dst2_ref[...] = jnp.where(first, d3, NPAD).reshape(ng, 1, CH)

    e1 = e_att + ef
    h = _ln(e1, fg_ref[...], fb_ref[...])
    h = _gelu(jnp.dot(h, w1_ref[...], preferred_element_type=jnp.float32) + b1_ref[...])
    eout_ref[...] = jnp.dot(h, w2_ref[...], preferred_element_type=jnp.float32) + b2_ref[...] + e1


def _tc2(qd, kvs, ef, dst3, We, be, Woe, boe, ln_e_g, ln_e_b, sel, selt,
         ffe_ln_g, ffe_ln_b, ffe_W1, ffe_b1, ffe_W2, ffe_b2):
    BE = 2000
    GB = BE // CH
    grid = E // BE
    full = lambda shape: pl.BlockSpec(shape, lambda i: (0, 0))
    return pl.pallas_call(
        _tc2_body,
        grid=(grid,),
        in_specs=[
            pl.BlockSpec((BE, D), lambda i: (i, 0)),
            pl.BlockSpec((BE, 2 * D), lambda i: (i, 0)),
            pl.BlockSpec((BE, DE), lambda i: (i, 0)),
            pl.BlockSpec((GB, 1, CH), lambda i: (i, 0, 0)),
            full((DE, D)), full((1, D)),
            full((D, DE)), full((1, DE)),
            full((1, DE)), full((1, DE)),
            full((D, H)), full((H, D)),
            full((1, DE)), full((1, DE)),
            full((DE, DEFF)), full((1, DEFF)),
            full((DEFF, DE)), full((1, DE)),
        ],
        out_specs=[
            pl.BlockSpec((BE, DE), lambda i: (i, 0)),
            pl.BlockSpec((BE, DW), lambda i: (i, 0)),
            pl.BlockSpec((GB, 1, CH), lambda i: (i, 0, 0)),
        ],
        out_shape=[
            jax.ShapeDtypeStruct((E, DE), jnp.float32),
            jax.ShapeDtypeStruct((E, DW), jnp.float32),
            jax.ShapeDtypeStruct((NG, 1, CH), jnp.int32),
        ],
    )(qd, kvs, ef, dst3, We, be, Woe, boe, ln_e_g, ln_e_b, sel, selt,
      ffe_ln_g, ffe_ln_b, ffe_W1, ffe_b1, ffe_W2, ffe_b2)


def _sc3_body(dst2_ref, wva_ref, zin_ref, oacc_ref, idxbuf, rowbuf, accbuf,
              zbuf, sem1, sem2):
    c = lax.axis_index("c")
    s = lax.axis_index("s")
    wid = s * 2 + c
    base = pl.multiple_of(wid * EPW, 8)
    gbase = wid * NCH
    pbase = wid * NPAD2

    pltpu.sync_copy(zin_ref, zbuf)

    def zrow(i, carry):
        pltpu.sync_copy(zbuf, oacc_ref.at[pl.ds(pbase + i * CH, CH)])
        return carry
    lax.fori_loop(0, NPAD2 // CH, zrow, 0)

    pltpu.sync_copy(dst2_ref.at[pl.ds(base, EPW)], idxbuf)

    def addb(v, carry):
        idxbuf[pl.ds(v * 16, 16)] = idxbuf[pl.ds(v * 16, 16)] + pbase
        return carry
    lax.fori_loop(0, EPW // 16, addb, 0)

    def chunk(i, carry):
        off = pl.multiple_of(base + i * CH, 8)
        cw = pltpu.async_copy(wva_ref.at[pl.ds(off, CH)], rowbuf, sem2)
        cg = pltpu.async_copy(oacc_ref.at[idxbuf.at[pl.ds(i * CH, CH)]],
                              accbuf, sem1)
        cw.wait()
        cg.wait()

        def rmw(r, cc):
            for j in range(DW // 16):
                accbuf[r, pl.ds(j * 16, 16)] = (
                    accbuf[r, pl.ds(j * 16, 16)] + rowbuf[r, pl.ds(j * 16, 16)])
            return cc
        lax.fori_loop(0, CH, rmw, 0)

        pltpu.sync_copy(accbuf, oacc_ref.at[idxbuf.at[pl.ds(i * CH, CH)]])
        return carry

    lax.fori_loop(0, NCH, chunk, 0)


def _sc3(dst2, wva, zin):
    mesh = plsc.VectorSubcoreMesh(core_axis_name="c", subcore_axis_name="s")
    f = functools.partial(
        pl.kernel,
        mesh=mesh,
        out_type=jax.ShapeDtypeStruct((NW * NPAD2, DW), jnp.float32),
        scratch_types=[
            pltpu.VMEM((EPW,), jnp.int32),
            pltpu.VMEM((CH, DW), jnp.float32),
            pltpu.VMEM((CH, DW), jnp.float32),
            pltpu.VMEM((CH, DW), jnp.float32),
            pltpu.SemaphoreType.DMA,
            pltpu.SemaphoreType.DMA,
        ],
    )(_sc3_body)
    return f(dst2, wva, zin)


def _tc2b_body(p_ref, acc_ref, accsc):
    w = pl.program_id(1)

    @pl.when(w == 0)
    def _():
        accsc[...] = p_ref[...]

    @pl.when(w > 0)
    def _():
        accsc[...] = accsc[...] + p_ref[...]

    @pl.when(w == NW - 1)
    def _():
        acc_ref[...] = accsc[...]


def _tc2b(parts):
    BN = 1280
    grid_i = NPAD // BN
    stride = NPAD2 // BN
    return pl.pallas_call(
        _tc2b_body,
        grid=(grid_i, NW),
        in_specs=[pl.BlockSpec((BN, DW), lambda i, w: (w * stride + i, 0))],
        out_specs=pl.BlockSpec((BN, DW), lambda i, w: (i, 0)),
        out_shape=jax.ShapeDtypeStruct((NPAD, DW), jnp.float32),
        scratch_shapes=[pltpu.VMEM((BN, DW), jnp.float32)],
    )(parts)


def _tc3_body(x_ref, acc_ref, wox_ref, box_ref, lng_ref, lnb_ref,
              w1_ref, b1_ref, w2_ref, b2_ref, out_ref):
    agg = acc_ref[:, :D]
    den = acc_ref[:, D:]
    xa = agg / (den + 1e-9)
    x_att = jnp.dot(xa, wox_ref[...], preferred_element_type=jnp.float32) + box_ref[...]
    x1 = x_att + x_ref[...]
    h = _ln(x1, lng_ref[...], lnb_ref[...])
    h = _gelu(jnp.dot(h, w1_ref[...], preferred_element_type=jnp.float32) + b1_ref[...])
    out_ref[...] = jnp.dot(h, w2_ref[...], preferred_element_type=jnp.float32) + b2_ref[...] + x1


def _tc3(xpad, acc, Wox, box, ffx_ln_g, ffx_ln_b, ffx_W1, ffx_b1, ffx_W2, ffx_b2):
    BN = 1280
    grid = NPAD // BN
    full = lambda shape: pl.BlockSpec(shape, lambda i: (0, 0))
    return pl.pallas_call(
        _tc3_body,
        grid=(grid,),
        in_specs=[
            pl.BlockSpec((BN, D), lambda i: (i, 0)),
            pl.BlockSpec((BN, DW), lambda i: (i, 0)),
            full((D, D)), full((1, D)),
            full((1, D)), full((1, D)),
            full((D, DFF)), full((1, DFF)),
            full((DFF, D)), full((1, D)),
        ],
        out_specs=pl.BlockSpec((BN, D), lambda i: (i, 0)),
        out_shape=jax.ShapeDtypeStruct((NPAD, D), jnp.float32),
    )(xpad, acc, Wox, box, ffx_ln_g, ffx_ln_b, ffx_W1, ffx_b1, ffx_W2, ffx_b2)


def kernel(x, edge_index, edge_features, Wq, bq, Wk, bk, Wv, bv, We, be,
           Wox, box, Woe, boe, ln_x_g, ln_x_b, ln_e_g, ln_e_b,
           ffx_ln_g, ffx_ln_b, ffx_W1, ffx_b1, ffx_W2, ffx_b2,
           ffe_ln_g, ffe_ln_b, ffe_W1, ffe_b1, ffe_W2, ffe_b2):
    src = edge_index[0]
    dst = edge_index[1]
    r = lambda v: v.reshape(1, -1)

    hd = jnp.arange(D, dtype=jnp.int32) // DH
    sel = (hd[:, None] == jnp.arange(H, dtype=jnp.int32)[None, :]).astype(jnp.float32)
    selt = sel.T

    q, kv = _tc1(x, r(ln_x_g), r(ln_x_b), Wq, r(bq), Wk, r(bk), Wv, r(bv))
    qd, kvs = _sc1(dst, src, q, kv)
    e_out, wva, dst2 = _tc2(
        qd, kvs, edge_features, dst.reshape(NG, 1, CH), We, r(be), Woe, r(boe),
        r(ln_e_g), r(ln_e_b), sel, selt,
        r(ffe_ln_g), r(ffe_ln_b), ffe_W1, r(ffe_b1), ffe_W2, r(ffe_b2))
    parts = _sc3(dst2.reshape(E), wva, jnp.zeros((CH, DW), jnp.float32))
    acc = _tc2b(parts)
    xpad = jnp.zeros((NPAD, D), jnp.float32).at[:N].set(x)
    x_out = _tc3(
        xpad, acc, Wox, r(box), r(ffx_ln_g), r(ffx_ln_b),
        ffx_W1, r(ffx_b1), ffx_W2, r(ffx_b2))[:N]
    return (x_out, e_out)


# speedup vs baseline: 22.3147x; 1.0121x over previous
import functools

import jax
import jax.numpy as jnp
from jax import lax
from jax.experimental import pallas as pl
from jax.experimental.pallas import tpu as pltpu
from jax.experimental.pallas import tpu_sc as plsc

N = 10000
E = 320000
D = 128
DE = 16
H = 8
DH = 16
DFF = 512
DEFF = 64
EPS = 1e-05

NPAD = 10240
NPAD2 = 11520
NW = 32
EPW = E // NW
CH = 80
NCH = EPW // CH
DW = 2 * D
NG = E // CH


def _ln(x, g, b):
    mu = jnp.mean(x, axis=-1, keepdims=True)
    xc = x - mu
    var = jnp.mean(xc * xc, axis=-1, keepdims=True)
    return xc * lax.rsqrt(var + EPS) * g + b


def _gelu(x):
    return x * 0.5 * (1.0 + lax.erf(x * 0.7071067811865476))


def _tc1_body(x_ref, g_ref, b_ref, wq_ref, bq_ref, wk_ref, bk_ref,
              wv_ref, bv_ref, q_ref, kv_ref):
    xn = _ln(x_ref[...], g_ref[...], b_ref[...])
    q_ref[...] = jnp.dot(xn, wq_ref[...], preferred_element_type=jnp.float32) + bq_ref[...]
    kv_ref[:, :D] = jnp.dot(xn, wk_ref[...], preferred_element_type=jnp.float32) + bk_ref[...]
    kv_ref[:, D:] = jnp.dot(xn, wv_ref[...], preferred_element_type=jnp.float32) + bv_ref[...]


def _tc1(x, ln_x_g, ln_x_b, Wq, bq, Wk, bk, Wv, bv):
    BN = 2000
    grid = N // BN
    full = lambda shape: pl.BlockSpec(shape, lambda i: (0, 0))
    return pl.pallas_call(
        _tc1_body,
        grid=(grid,),
        in_specs=[
            pl.BlockSpec((BN, D), lambda i: (i, 0)),
            full((1, D)), full((1, D)),
            full((D, D)), full((1, D)),
            full((D, D)), full((1, D)),
            full((D, D)), full((1, D)),
        ],
        out_specs=[
            pl.BlockSpec((BN, D), lambda i: (i, 0)),
            pl.BlockSpec((BN, 2 * D), lambda i: (i, 0)),
        ],
        out_shape=[
            jax.ShapeDtypeStruct((N, D), jnp.float32),
            jax.ShapeDtypeStruct((N, 2 * D), jnp.float32),
        ],
    )(x, ln_x_g, ln_x_b, Wq, bq, Wk, bk, Wv, bv)


def _sc1_body(dst_ref, src_ref, q_ref, kv_ref, qd_ref, kvs_ref,
              idxd, idxs, qbuf, kvbuf, sem1, sem2, sem3):
    wid = lax.axis_index("s") * 2 + lax.axis_index("c")
    base = pl.multiple_of(wid * EPW, 8)

    pltpu.sync_copy(dst_ref.at[pl.ds(base, EPW)], idxd)
    pltpu.sync_copy(src_ref.at[pl.ds(base, EPW)], idxs)

    def chunk(i, carry):
        off = pl.multiple_of(base + i * CH, 8)
        c1 = pltpu.async_copy(q_ref.at[idxd.at[pl.ds(i * CH, CH)]], qbuf, sem1)
        c2 = pltpu.async_copy(kv_ref.at[idxs.at[pl.ds(i * CH, CH)]], kvbuf, sem2)
        c1.wait()
        w1 = pltpu.async_copy(qbuf, qd_ref.at[pl.ds(off, CH)], sem3)
        c2.wait()
        pltpu.sync_copy(kvbuf, kvs_ref.at[pl.ds(off, CH)])
        w1.wait()
        return carry

    lax.fori_loop(0, NCH, chunk, 0)


def _sc1(dst, src, q, kv):
    mesh = plsc.VectorSubcoreMesh(core_axis_name="c", subcore_axis_name="s")
    f = functools.partial(
        pl.kernel,
        mesh=mesh,
        out_type=[
            jax.ShapeDtypeStruct((E, D), jnp.float32),
            jax.ShapeDtypeStruct((E, 2 * D), jnp.float32),
        ],
        scratch_types=[
            pltpu.VMEM((EPW,), jnp.int32),
            pltpu.VMEM((EPW,), jnp.int32),
            pltpu.VMEM((CH, D), jnp.float32),
            pltpu.VMEM((CH, 2 * D), jnp.float32),
            pltpu.SemaphoreType.DMA,
            pltpu.SemaphoreType.DMA,
            pltpu.SemaphoreType.DMA,
        ],
    )(_sc1_body)
    return f(dst, src, q, kv)


def _tc2_body(qd_ref, kvs_ref, ef_ref, dst_ref, we_ref, be_ref, woe_ref, boe_ref,
              lng_ref, lnb_ref, sel_ref, selt_ref,
              fg_ref, fb_ref, w1_ref, b1_ref, w2_ref, b2_ref,
              eout_ref, wva_ref, dst2_ref):
    ks = kvs_ref[:, :D]
    vs = kvs_ref[:, D:]
    ef = ef_ref[...]
    en = _ln(ef, lng_ref[...], lnb_ref[...])
    ep = jnp.dot(en, we_ref[...], preferred_element_type=jnp.float32) + be_ref[...]
    score = qd_ref[...] * ks * ep * 0.25
    e_att = jnp.dot(score, woe_ref[...], preferred_element_type=jnp.float32) + boe_ref[...]
    logits = jnp.dot(score, sel_ref[...], preferred_element_type=jnp.float32)
    a = jnp.exp(logits)
    ab = jnp.dot(a, selt_ref[...], preferred_element_type=jnp.float32)
    wva = jnp.concatenate([ab * vs, ab], axis=1)

    ng = wva.shape[0] // CH
    d3 = dst_ref[...].reshape(-1, CH)
    eq = (d3[:, :, None] == d3[:, None, :])
    eqf = eq.astype(jnp.float32)
    ii = lax.broadcasted_iota(jnp.int32, (CH, CH), 0)
    jj = lax.broadcasted_iota(jnp.int32, (CH, CH), 1)
    tm = (jj < ii).astype(jnp.float32)
    before = jnp.sum(eqf * tm[None], axis=2)
    first = before == 0.0
    rows3 = wva.reshape(ng, CH, DW)
    comb = lax.dot_general(eqf, rows3, (((2,), (1,)), ((0,), (0,))),
                           preferred_element_type=jnp.float32)
    rows3 = jnp.where(first[:, :, None], comb, 0.0)
    wva_ref[...] = rows3.reshape(ng * CH, DW)
    dst2_ref[...] = jnp.where(first, d3, NPAD).reshape(ng, 1, CH)

    e1 = e_att + ef
    h = _ln(e1, fg_ref[...], fb_ref[...])
    h = _gelu(jnp.dot(h, w1_ref[...], preferred_element_type=jnp.float32) + b1_ref[...])
    eout_ref[...] = jnp.dot(h, w2_ref[...], preferred_element_type=jnp.float32) + b2_ref[...] + e1


def _tc2(qd, kvs, ef, dst3, We, be, Woe, boe, ln_e_g, ln_e_b, sel, selt,
         ffe_ln_g, ffe_ln_b, ffe_W1, ffe_b1, ffe_W2, ffe_b2):
    BE = 2000
    GB = BE // CH
    grid = E // BE
    full = lambda shape: pl.BlockSpec(shape, lambda i: (0, 0))
    return pl.pallas_call(
        _tc2_body,
        grid=(grid,),
        in_specs=[
            pl.BlockSpec((BE, D), lambda i: (i, 0)),
            pl.BlockSpec((BE, 2 * D), lambda i: (i, 0)),
            pl.BlockSpec((BE, DE), lambda i: (i, 0)),
            pl.BlockSpec((GB, 1, CH), lambda i: (i, 0, 0)),
            full((DE, D)), full((1, D)),
            full((D, DE)), full((1, DE)),
            full((1, DE)), full((1, DE)),
            full((D, H)), full((H, D)),
            full((1, DE)), full((1, DE)),
            full((DE, DEFF)), full((1, DEFF)),
            full((DEFF, DE)), full((1, DE)),
        ],
        out_specs=[
            pl.BlockSpec((BE, DE), lambda i: (i, 0)),
            pl.BlockSpec((BE, DW), lambda i: (i, 0)),
            pl.BlockSpec((GB, 1, CH), lambda i: (i, 0, 0)),
        ],
        out_shape=[
            jax.ShapeDtypeStruct((E, DE), jnp.float32),
            jax.ShapeDtypeStruct((E, DW), jnp.float32),
            jax.ShapeDtypeStruct((NG, 1, CH), jnp.int32),
        ],
    )(qd, kvs, ef, dst3, We, be, Woe, boe, ln_e_g, ln_e_b, sel, selt,
      ffe_ln_g, ffe_ln_b, ffe_W1, ffe_b1, ffe_W2, ffe_b2)


def _sc3_body(dst2_ref, wva_ref, zin_ref, oacc_ref, idxbuf, rowbuf, rowbuf2,
              accbuf, zbuf, sem1, sem2):
    c = lax.axis_index("c")
    s = lax.axis_index("s")
    wid = s * 2 + c
    base = pl.multiple_of(wid * EPW, 8)
    gbase = wid * NCH
    pbase = wid * NPAD2

    pltpu.sync_copy(zin_ref, zbuf)

    def zrow(i, carry):
        pltpu.async_copy(zbuf, oacc_ref.at[pl.ds(pbase + i * CH, CH)], sem2)
        return carry
    lax.fori_loop(0, NPAD2 // CH, zrow, 0)

    def zdrain(i, carry):
        pltpu.make_async_copy(zin_ref, zbuf, sem2).wait()
        return carry
    lax.fori_loop(0, NPAD2 // CH, zdrain, 0)

    pltpu.sync_copy(dst2_ref.at[pl.ds(base, EPW)], idxbuf)

    def addb(v, carry):
        idxbuf[pl.ds(v * 16, 16)] = idxbuf[pl.ds(v * 16, 16)] + pbase
        return carry
    lax.fori_loop(0, EPW // 16, addb, 0)

    def chunk(i, carry):
        off = pl.multiple_of(base + i * CH, 8)
        cw = pltpu.async_copy(wva_ref.at[pl.ds(off, CH)], rowbuf, sem2)
        cg = pltpu.async_copy(oacc_ref.at[idxbuf.at[pl.ds(i * CH, CH)]],
                              accbuf, sem1)
        cw.wait()
        cg.wait()

        def rmw(r, cc):
            for j in range(DW // 16):
                accbuf[r, pl.ds(j * 16, 16)] = (
                    accbuf[r, pl.ds(j * 16, 16)] + rowbuf[r, pl.ds(j * 16, 16)])
            return cc
        lax.fori_loop(0, CH, rmw, 0)

        pltpu.sync_copy(accbuf, oacc_ref.at[idxbuf.at[pl.ds(i * CH, CH)]])
        return carry

    lax.fori_loop(0, NCH, chunk, 0)


def _sc3(dst2, wva, zin):
    mesh = plsc.VectorSubcoreMesh(core_axis_name="c", subcore_axis_name="s")
    f = functools.partial(
        pl.kernel,
        mesh=mesh,
        out_type=jax.ShapeDtypeStruct((NW * NPAD2, DW), jnp.float32),
        scratch_types=[
            pltpu.VMEM((EPW,), jnp.int32),
            pltpu.VMEM((CH, DW), jnp.float32),
            pltpu.VMEM((CH, DW), jnp.float32),
            pltpu.VMEM((CH, DW), jnp.float32),
            pltpu.VMEM((CH, DW), jnp.float32),
            pltpu.SemaphoreType.DMA,
            pltpu.SemaphoreType.DMA,
        ],
    )(_sc3_body)
    return f(dst2, wva, zin)


def _tc2b_body(p_ref, acc_ref, accsc):
    w = pl.program_id(1)

    @pl.when(w == 0)
    def _():
        accsc[...] = p_ref[...]

    @pl.when(w > 0)
    def _():
        accsc[...] = accsc[...] + p_ref[...]

    @pl.when(w == NW - 1)
    def _():
        acc_ref[...] = accsc[...]


def _tc2b(parts):
    BN = 1280
    grid_i = NPAD // BN
    stride = NPAD2 // BN
    return pl.pallas_call(
        _tc2b_body,
        grid=(grid_i, NW),
        in_specs=[pl.BlockSpec((BN, DW), lambda i, w: (w * stride + i, 0))],
        out_specs=pl.BlockSpec((BN, DW), lambda i, w: (i, 0)),
        out_shape=jax.ShapeDtypeStruct((NPAD, DW), jnp.float32),
        scratch_shapes=[pltpu.VMEM((BN, DW), jnp.float32)],
    )(parts)


def _tc3_body(x_ref, acc_ref, wox_ref, box_ref, lng_ref, lnb_ref,
              w1_ref, b1_ref, w2_ref, b2_ref, out_ref):
    agg = acc_ref[:, :D]
    den = acc_ref[:, D:]
    xa = agg / (den + 1e-9)
    x_att = jnp.dot(xa, wox_ref[...], preferred_element_type=jnp.float32) + box_ref[...]
    x1 = x_att + x_ref[...]
    h = _ln(x1, lng_ref[...], lnb_ref[...])
    h = _gelu(jnp.dot(h, w1_ref[...], preferred_element_type=jnp.float32) + b1_ref[...])
    out_ref[...] = jnp.dot(h, w2_ref[...], preferred_element_type=jnp.float32) + b2_ref[...] + x1


def _tc3(xpad, acc, Wox, box, ffx_ln_g, ffx_ln_b, ffx_W1, ffx_b1, ffx_W2, ffx_b2):
    BN = 1280
    grid = NPAD // BN
    full = lambda shape: pl.BlockSpec(shape, lambda i: (0, 0))
    return pl.pallas_call(
        _tc3_body,
        grid=(grid,),
        in_specs=[
            pl.BlockSpec((BN, D), lambda i: (i, 0)),
            pl.BlockSpec((BN, DW), lambda i: (i, 0)),
            full((D, D)), full((1, D)),
            full((1, D)), full((1, D)),
            full((D, DFF)), full((1, DFF)),
            full((DFF, D)), full((1, D)),
        ],
        out_specs=pl.BlockSpec((BN, D), lambda i: (i, 0)),
        out_shape=jax.ShapeDtypeStruct((NPAD, D), jnp.float32),
    )(xpad, acc, Wox, box, ffx_ln_g, ffx_ln_b, ffx_W1, ffx_b1, ffx_W2, ffx_b2)


def kernel(x, edge_index, edge_features, Wq, bq, Wk, bk, Wv, bv, We, be,
           Wox, box, Woe, boe, ln_x_g, ln_x_b, ln_e_g, ln_e_b,
           ffx_ln_g, ffx_ln_b, ffx_W1, ffx_b1, ffx_W2, ffx_b2,
           ffe_ln_g, ffe_ln_b, ffe_W1, ffe_b1, ffe_W2, ffe_b2):
    src = edge_index[0]
    dst = edge_index[1]
    r = lambda v: v.reshape(1, -1)

    hd = jnp.arange(D, dtype=jnp.int32) // DH
    sel = (hd[:, None] == jnp.arange(H, dtype=jnp.int32)[None, :]).astype(jnp.float32)
    selt = sel.T

    q, kv = _tc1(x, r(ln_x_g), r(ln_x_b), Wq, r(bq), Wk, r(bk), Wv, r(bv))
    qd, kvs = _sc1(dst, src, q, kv)
    e_out, wva, dst2 = _tc2(
        qd, kvs, edge_features, dst.reshape(NG, 1, CH), We, r(be), Woe, r(boe),
        r(ln_e_g), r(ln_e_b), sel, selt,
        r(ffe_ln_g), r(ffe_ln_b), ffe_W1, r(ffe_b1), ffe_W2, r(ffe_b2))
    parts = _sc3(dst2.reshape(E), wva, jnp.zeros((CH, DW), jnp.float32))
    acc = _tc2b(parts)
    xpad = jnp.zeros((NPAD, D), jnp.float32).at[:N].set(x)
    x_out = _tc3(
        xpad, acc, Wox, r(box), r(ffx_ln_g), r(ffx_ln_b),
        ffx_W1, r(ffx_b1), ffx_W2, r(ffx_b2))[:N]
    return (x_out, e_out)


# speedup vs baseline: 22.5088x; 1.0087x over previous
import functools

import jax
import jax.numpy as jnp
from jax import lax
from jax.experimental import pallas as pl
from jax.experimental.pallas import tpu as pltpu
from jax.experimental.pallas import tpu_sc as plsc

N = 10000
E = 320000
D = 128
DE = 16
H = 8
DH = 16
DFF = 512
DEFF = 64
EPS = 1e-05

NPAD = 10240
NPAD2 = 11520
NW = 32
EPW = E // NW
CH = 80
NCH = EPW // CH
CH1 = 128
NCH1 = EPW // CH1
TAIL1 = EPW - NCH1 * CH1
DW = 2 * D
NG = E // CH


def _ln(x, g, b):
    mu = jnp.mean(x, axis=-1, keepdims=True)
    xc = x - mu
    var = jnp.mean(xc * xc, axis=-1, keepdims=True)
    return xc * lax.rsqrt(var + EPS) * g + b


def _gelu(x):
    return x * 0.5 * (1.0 + lax.erf(x * 0.7071067811865476))


def _tc1_body(x_ref, g_ref, b_ref, wq_ref, bq_ref, wk_ref, bk_ref,
              wv_ref, bv_ref, q_ref, kv_ref):
    xn = _ln(x_ref[...], g_ref[...], b_ref[...])
    q_ref[...] = jnp.dot(xn, wq_ref[...], preferred_element_type=jnp.float32) + bq_ref[...]
    kv_ref[:, :D] = jnp.dot(xn, wk_ref[...], preferred_element_type=jnp.float32) + bk_ref[...]
    kv_ref[:, D:] = jnp.dot(xn, wv_ref[...], preferred_element_type=jnp.float32) + bv_ref[...]


def _tc1(x, ln_x_g, ln_x_b, Wq, bq, Wk, bk, Wv, bv):
    BN = 2000
    grid = N // BN
    full = lambda shape: pl.BlockSpec(shape, lambda i: (0, 0))
    return pl.pallas_call(
        _tc1_body,
        grid=(grid,),
        in_specs=[
            pl.BlockSpec((BN, D), lambda i: (i, 0)),
            full((1, D)), full((1, D)),
            full((D, D)), full((1, D)),
            full((D, D)), full((1, D)),
            full((D, D)), full((1, D)),
        ],
        out_specs=[
            pl.BlockSpec((BN, D), lambda i: (i, 0)),
            pl.BlockSpec((BN, 2 * D), lambda i: (i, 0)),
        ],
        out_shape=[
            jax.ShapeDtypeStruct((N, D), jnp.float32),
            jax.ShapeDtypeStruct((N, 2 * D), jnp.float32),
        ],
    )(x, ln_x_g, ln_x_b, Wq, bq, Wk, bk, Wv, bv)


def _sc1_body(dst_ref, src_ref, q_ref, kv_ref, qd_ref, kvs_ref,
              idxd, idxs, qbuf, kvbuf, sem1, sem2, sem3):
    wid = lax.axis_index("s") * 2 + lax.axis_index("c")
    base = pl.multiple_of(wid * EPW, 8)

    pltpu.sync_copy(dst_ref.at[pl.ds(base, EPW)], idxd)
    pltpu.sync_copy(src_ref.at[pl.ds(base, EPW)], idxs)

    def chunk(i, carry):
        off = pl.multiple_of(base + i * CH1, 8)
        c1 = pltpu.async_copy(q_ref.at[idxd.at[pl.ds(i * CH1, CH1)]], qbuf, sem1)
        c2 = pltpu.async_copy(kv_ref.at[idxs.at[pl.ds(i * CH1, CH1)]], kvbuf, sem2)
        c1.wait()
        w1 = pltpu.async_copy(qbuf, qd_ref.at[pl.ds(off, CH1)], sem3)
        c2.wait()
        pltpu.sync_copy(kvbuf, kvs_ref.at[pl.ds(off, CH1)])
        w1.wait()
        return carry

    lax.fori_loop(0, NCH1, chunk, 0)

    toff = pl.multiple_of(base + NCH1 * CH1, 8)
    t1 = pltpu.async_copy(q_ref.at[idxd.at[pl.ds(NCH1 * CH1, TAIL1)]],
                          qbuf.at[pl.ds(0, TAIL1)], sem1)
    t2 = pltpu.async_copy(kv_ref.at[idxs.at[pl.ds(NCH1 * CH1, TAIL1)]],
                          kvbuf.at[pl.ds(0, TAIL1)], sem2)
    t1.wait()
    t2.wait()
    pltpu.sync_copy(qbuf.at[pl.ds(0, TAIL1)], qd_ref.at[pl.ds(toff, TAIL1)])
    pltpu.sync_copy(kvbuf.at[pl.ds(0, TAIL1)], kvs_ref.at[pl.ds(toff, TAIL1)])


def _sc1(dst, src, q, kv):
    mesh = plsc.VectorSubcoreMesh(core_axis_name="c", subcore_axis_name="s")
    f = functools.partial(
        pl.kernel,
        mesh=mesh,
        out_type=[
            jax.ShapeDtypeStruct((E, D), jnp.float32),
            jax.ShapeDtypeStruct((E, 2 * D), jnp.float32),
        ],
        scratch_types=[
            pltpu.VMEM((EPW,), jnp.int32),
            pltpu.VMEM((EPW,), jnp.int32),
            pltpu.VMEM((CH1, D), jnp.float32),
            pltpu.VMEM((CH1, 2 * D), jnp.float32),
            pltpu.SemaphoreType.DMA,
            pltpu.SemaphoreType.DMA,
            pltpu.SemaphoreType.DMA,
        ],
    )(_sc1_body)
    return f(dst, src, q, kv)


def _tc2_body(qd_ref, kvs_ref, ef_ref, dst_ref, we_ref, be_ref, woe_ref, boe_ref,
              lng_ref, lnb_ref, sel_ref, selt_ref,
              fg_ref, fb_ref, w1_ref, b1_ref, w2_ref, b2_ref,
              eout_ref, wva_ref, dst2_ref):
    ks = kvs_ref[:, :D]
    vs = kvs_ref[:, D:]
    ef = ef_ref[...]
    en = _ln(ef, lng_ref[...], lnb_ref[...])
    ep = jnp.dot(en, we_ref[...], preferred_element_type=jnp.float32) + be_ref[...]
    score = qd_ref[...] * ks * ep * 0.25
    e_att = jnp.dot(score, woe_ref[...], preferred_element_type=jnp.float32) + boe_ref[...]
    logits = jnp.dot(score, sel_ref[...], preferred_element_type=jnp.float32)
    a = jnp.exp(logits)
    ab = jnp.dot(a, selt_ref[...], preferred_element_type=jnp.float32)
    wva = jnp.concatenate([ab * vs, ab], axis=1)

    ng = wva.shape[0] // CH
    d3 = dst_ref[...].reshape(-1, CH)
    eq = (d3[:, :, None] == d3[:, None, :])
    eqf = eq.astype(jnp.float32)
    ii = lax.broadcasted_iota(jnp.int32, (CH, CH), 0)
    jj = lax.broadcasted_iota(jnp.int32, (CH, CH), 1)
    tm = (jj < ii).astype(jnp.float32)
    before = jnp.sum(eqf * tm[None], axis=2)
    first = before == 0.0
    rows3 = wva.reshape(ng, CH, DW)
    comb = lax.dot_general(eqf, rows3, (((2,), (1,)), ((0,), (0,))),
                           preferred_element_type=jnp.float32)
    rows3 = jnp.where(first[:, :, None], comb, 0.0)
    wva_ref[...] = rows3.reshape(ng * CH, DW)
    dst2_ref[...] = jnp.where(first, d3, NPAD).reshape(ng, 1, CH)

    e1 = e_att + ef
    h = _ln(e1, fg_ref[...], fb_ref[...])
    h = _gelu(jnp.dot(h, w1_ref[...], preferred_element_type=jnp.float32) + b1_ref[...])
    eout_ref[...] = jnp.dot(h, w2_ref[...], preferred_element_type=jnp.float32) + b2_ref[...] + e1


def _tc2(qd, kvs, ef, dst3, We, be, Woe, boe, ln_e_g, ln_e_b, sel, selt,
         ffe_ln_g, ffe_ln_b, ffe_W1, ffe_b1, ffe_W2, ffe_b2):
    BE = 2000
    GB = BE // CH
    grid = E // BE
    full = lambda shape: pl.BlockSpec(shape, lambda i: (0, 0))
    return pl.pallas_call(
        _tc2_body,
        grid=(grid,),
        in_specs=[
            pl.BlockSpec((BE, D), lambda i: (i, 0)),
            pl.BlockSpec((BE, 2 * D), lambda i: (i, 0)),
            pl.BlockSpec((BE, DE), lambda i: (i, 0)),
            pl.BlockSpec((GB, 1, CH), lambda i: (i, 0, 0)),
            full((DE, D)), full((1, D)),
            full((D, DE)), full((1, DE)),
            full((1, DE)), full((1, DE)),
            full((D, H)), full((H, D)),
            full((1, DE)), full((1, DE)),
            full((DE, DEFF)), full((1, DEFF)),
            full((DEFF, DE)), full((1, DE)),
        ],
        out_specs=[
            pl.BlockSpec((BE, DE), lambda i: (i, 0)),
            pl.BlockSpec((BE, DW), lambda i: (i, 0)),
            pl.BlockSpec((GB, 1, CH), lambda i: (i, 0, 0)),
        ],
        out_shape=[
            jax.ShapeDtypeStruct((E, DE), jnp.float32),
            jax.ShapeDtypeStruct((E, DW), jnp.float32),
            jax.ShapeDtypeStruct((NG, 1, CH), jnp.int32),
        ],
    )(qd, kvs, ef, dst3, We, be, Woe, boe, ln_e_g, ln_e_b, sel, selt,
      ffe_ln_g, ffe_ln_b, ffe_W1, ffe_b1, ffe_W2, ffe_b2)


def _sc3_body(dst2_ref, wva_ref, zin_ref, oacc_ref, idxbuf, rowbuf, rowbuf2,
              accbuf, zbuf, sem1, sem2):
    c = lax.axis_index("c")
    s = lax.axis_index("s")
    wid = s * 2 + c
    base = pl.multiple_of(wid * EPW, 8)
    gbase = wid * NCH
    pbase = wid * NPAD2

    pltpu.sync_copy(zin_ref, zbuf)

    def zrow(i, carry):
        pltpu.async_copy(zbuf, oacc_ref.at[pl.ds(pbase + i * CH, CH)], sem2)
        return carry
    lax.fori_loop(0, NPAD2 // CH, zrow, 0)

    def zdrain(i, carry):
        pltpu.make_async_copy(zin_ref, zbuf, sem2).wait()
        return carry
    lax.fori_loop(0, NPAD2 // CH, zdrain, 0)

    pltpu.sync_copy(dst2_ref.at[pl.ds(base, EPW)], idxbuf)

    def addb(v, carry):
        idxbuf[pl.ds(v * 16, 16)] = idxbuf[pl.ds(v * 16, 16)] + pbase
        return carry
    lax.fori_loop(0, EPW // 16, addb, 0)

    def chunk(i, carry):
        off = pl.multiple_of(base + i * CH, 8)
        cw = pltpu.async_copy(wva_ref.at[pl.ds(off, CH)], rowbuf, sem2)
        cg = pltpu.async_copy(oacc_ref.at[idxbuf.at[pl.ds(i * CH, CH)]],
                              accbuf, sem1)
        cw.wait()
        cg.wait()

        def rmw(r, cc):
            for j in range(DW // 16):
                accbuf[r, pl.ds(j * 16, 16)] = (
                    accbuf[r, pl.ds(j * 16, 16)] + rowbuf[r, pl.ds(j * 16, 16)])
            return cc
        lax.fori_loop(0, CH, rmw, 0)

        pltpu.sync_copy(accbuf, oacc_ref.at[idxbuf.at[pl.ds(i * CH, CH)]])
        return carry

    lax.fori_loop(0, NCH, chunk, 0)


def _sc3(dst2, wva, zin):
    mesh = plsc.VectorSubcoreMesh(core_axis_name="c", subcore_axis_name="s")
    f = functools.partial(
        pl.kernel,
        mesh=mesh,
        out_type=jax.ShapeDtypeStruct((NW * NPAD2, DW), jnp.float32),
        scratch_types=[
            pltpu.VMEM((EPW,), jnp.int32),
            pltpu.VMEM((CH, DW), jnp.float32),
            pltpu.VMEM((CH, DW), jnp.float32),
            pltpu.VMEM((CH, DW), jnp.float32),
            pltpu.VMEM((CH, DW), jnp.float32),
            pltpu.SemaphoreType.DMA,
            pltpu.SemaphoreType.DMA,
        ],
    )(_sc3_body)
    return f(dst2, wva, zin)


def _tc2b_body(p_ref, acc_ref, accsc):
    w = pl.program_id(1)

    @pl.when(w == 0)
    def _():
        accsc[...] = p_ref[...]

    @pl.when(w > 0)
    def _():
        accsc[...] = accsc[...] + p_ref[...]

    @pl.when(w == NW - 1)
    def _():
        acc_ref[...] = accsc[...]


def _tc2b(parts):
    BN = 1280
    grid_i = NPAD // BN
    stride = NPAD2 // BN
    return pl.pallas_call(
        _tc2b_body,
        grid=(grid_i, NW),
        in_specs=[pl.BlockSpec((BN, DW), lambda i, w: (w * stride + i, 0))],
        out_specs=pl.BlockSpec((BN, DW), lambda i, w: (i, 0)),
        out_shape=jax.ShapeDtypeStruct((NPAD, DW), jnp.float32),
        scratch_shapes=[pltpu.VMEM((BN, DW), jnp.float32)],
    )(parts)


def _tc3_body(x_ref, acc_ref, wox_ref, box_ref, lng_ref, lnb_ref,
              w1_ref, b1_ref, w2_ref, b2_ref, out_ref):
    agg = acc_ref[:, :D]
    den = acc_ref[:, D:]
    xa = agg / (den + 1e-9)
    x_att = jnp.dot(xa, wox_ref[...], preferred_element_type=jnp.float32) + box_ref[...]
    x1 = x_att + x_ref[...]
    h = _ln(x1, lng_ref[...], lnb_ref[...])
    h = _gelu(jnp.dot(h, w1_ref[...], preferred_element_type=jnp.float32) + b1_ref[...])
    out_ref[...] = jnp.dot(h, w2_ref[...], preferred_element_type=jnp.float32) + b2_ref[...] + x1


def _tc3(xpad, acc, Wox, box, ffx_ln_g, ffx_ln_b, ffx_W1, ffx_b1, ffx_W2, ffx_b2):
    BN = 1280
    grid = NPAD // BN
    full = lambda shape: pl.BlockSpec(shape, lambda i: (0, 0))
    return pl.pallas_call(
        _tc3_body,
        grid=(grid,),
        in_specs=[
            pl.BlockSpec((BN, D), lambda i: (i, 0)),
            pl.BlockSpec((BN, DW), lambda i: (i, 0)),
            full((D, D)), full((1, D)),
            full((1, D)), full((1, D)),
            full((D, DFF)), full((1, DFF)),
            full((DFF, D)), full((1, D)),
        ],
        out_specs=pl.BlockSpec((BN, D), lambda i: (i, 0)),
        out_shape=jax.ShapeDtypeStruct((NPAD, D), jnp.float32),
    )(xpad, acc, Wox, box, ffx_ln_g, ffx_ln_b, ffx_W1, ffx_b1, ffx_W2, ffx_b2)


def kernel(x, edge_index, edge_features, Wq, bq, Wk, bk, Wv, bv, We, be,
           Wox, box, Woe, boe, ln_x_g, ln_x_b, ln_e_g, ln_e_b,
           ffx_ln_g, ffx_ln_b, ffx_W1, ffx_b1, ffx_W2, ffx_b2,
           ffe_ln_g, ffe_ln_b, ffe_W1, ffe_b1, ffe_W2, ffe_b2):
    src = edge_index[0]
    dst = edge_index[1]
    r = lambda v: v.reshape(1, -1)

    hd = jnp.arange(D, dtype=jnp.int32) // DH
    sel = (hd[:, None] == jnp.arange(H, dtype=jnp.int32)[None, :]).astype(jnp.float32)
    selt = sel.T

    q, kv = _tc1(x, r(ln_x_g), r(ln_x_b), Wq, r(bq), Wk, r(bk), Wv, r(bv))
    qd, kvs = _sc1(dst, src, q, kv)
    e_out, wva, dst2 = _tc2(
        qd, kvs, edge_features, dst.reshape(NG, 1, CH), We, r(be), Woe, r(boe),
        r(ln_e_g), r(ln_e_b), sel, selt,
        r(ffe_ln_g), r(ffe_ln_b), ffe_W1, r(ffe_b1), ffe_W2, r(ffe_b2))
    parts = _sc3(dst2.reshape(E), wva, jnp.zeros((CH, DW), jnp.float32))
    acc = _tc2b(parts)
    xpad = jnp.zeros((NPAD, D), jnp.float32).at[:N].set(x)
    x_out = _tc3(
        xpad, acc, Wox, r(box), r(ffx_ln_g), r(ffx_ln_b),
        ffx_W1, r(ffx_b1), ffx_W2, r(ffx_b2))[:N]
    return (x_out, e_out)
